# Initial kernel scaffold; baseline (speedup 1.0000x reference)
import jax
import jax.numpy as jnp
from jax.experimental import pallas as pl


def kernel(int_ids, species_emb, move_emb, ability_emb, item_emb, last_move_emb):
    raise NotImplementedError("write your pallas kernel here")


# speedup vs baseline: 9.2141x; 9.2141x over previous
import functools

import jax
import jax.numpy as jnp
from jax import lax
from jax.experimental import pallas as pl
from jax.experimental.pallas import tpu as pltpu
from jax.experimental.pallas import tpu_sc as plsc

BATCH = 4096
N_TOKENS = 50
TOKENS = BATCH * N_TOKENS
CHUNK = 128
NCHUNKS = TOKENS // CHUNK
NW = 32
CPW = NCHUNKS // NW
D_OUT = 256

FIELDS = (
    (0, 0, 64, 0),
    (1, 64, 32, 1),
    (2, 96, 32, 1),
    (3, 128, 32, 1),
    (4, 160, 32, 1),
    (5, 192, 16, 2),
    (6, 208, 16, 3),
    (7, 224, 32, 4),
)


def _body(idx_hbm, sp_hbm, mv_hbm, ab_hbm, it_hbm, lm_hbm, out_hbm,
          idxv, b0, b1, b2, b3, b4, b5, b6, b7, gsem, wsem):
    tables = (sp_hbm, mv_hbm, ab_hbm, it_hbm, lm_hbm)
    bufs = (b0, b1, b2, b3, b4, b5, b6, b7)
    wid = lax.axis_index("s") * 2 + lax.axis_index("c")

    def step(j, carry):
        g = wid * CPW + j
        tok = pl.multiple_of(g * CHUNK, CHUNK)
        pltpu.sync_copy(idx_hbm.at[g], idxv)
        gathers = []
        for i, (row, _, _, targ) in enumerate(FIELDS):
            gathers.append(
                pltpu.async_copy(tables[targ].at[idxv.at[row]], bufs[i], gsem))
        for c in gathers:
            c.wait()
        writes = []
        for i, (_, col, w, _) in enumerate(FIELDS):
            writes.append(
                pltpu.async_copy(
                    bufs[i], out_hbm.at[pl.ds(tok, CHUNK), pl.ds(col, w)], wsem))
        for c in writes:
            c.wait()
        return carry

    lax.fori_loop(0, CPW, step, 0)


@functools.partial(jax.jit, static_argnames=())
def _run(idx, sp, mv, ab, it, lm):
    scratch = [
        pltpu.VMEM((8, CHUNK), jnp.int32),
    ]
    for _, _, w, _ in FIELDS:
        scratch.append(pltpu.VMEM((CHUNK, w), jnp.float32))
    scratch += [pltpu.SemaphoreType.DMA, pltpu.SemaphoreType.DMA]
    kern = pl.kernel(
        _body,
        out_type=jax.ShapeDtypeStruct((TOKENS, D_OUT), jnp.float32),
        mesh=plsc.VectorSubcoreMesh(core_axis_name="c", subcore_axis_name="s"),
        scratch_types=scratch,
        compiler_params=pltpu.CompilerParams(use_tc_tiling_on_sc=False),
    )
    return kern(idx, sp, mv, ab, it, lm)


def kernel(int_ids, species_emb, move_emb, ability_emb, item_emb, last_move_emb):
    ids = int_ids.astype(jnp.int32)
    idx = ids.reshape(NCHUNKS, CHUNK, 8).transpose(0, 2, 1)
    out = _run(idx, species_emb, move_emb, ability_emb, item_emb, last_move_emb)
    return out.reshape(BATCH, N_TOKENS, D_OUT)


# speedup vs baseline: 9.8571x; 1.0698x over previous
import functools

import jax
import jax.numpy as jnp
from jax import lax
from jax.experimental import pallas as pl
from jax.experimental.pallas import tpu as pltpu
from jax.experimental.pallas import tpu_sc as plsc

BATCH = 4096
N_TOKENS = 50
TOKENS = BATCH * N_TOKENS
CHUNK = 128
NCHUNKS = TOKENS // CHUNK
NW = 32
CPW = NCHUNKS // NW
D_OUT = 256

FIELDS = (
    (0, 0, 64, 0),
    (1, 64, 32, 1),
    (2, 96, 32, 1),
    (3, 128, 32, 1),
    (4, 160, 32, 1),
    (5, 192, 16, 2),
    (6, 208, 16, 3),
    (7, 224, 32, 4),
)


def _body(idx_hbm, sp_hbm, mv_hbm, ab_hbm, it_hbm, lm_hbm, out_hbm,
          idxv, bufs0, bufs1, gsem, wsem0, wsem1, isem):
    tables = (sp_hbm, mv_hbm, ab_hbm, it_hbm, lm_hbm)
    bufs = (bufs0, bufs1)
    wsems = (wsem0, wsem1)
    wid = lax.axis_index("s") * 2 + lax.axis_index("c")
    chunk0 = wid * CPW

    def fire_gathers(b, g):
        for i, (row, _, _, targ) in enumerate(FIELDS):
            pltpu.async_copy(tables[targ].at[idxv.at[b, row]], bufs[b][i], gsem)

    def wait_gathers(b):
        for i, (row, _, _, targ) in enumerate(FIELDS):
            pltpu.make_async_copy(tables[targ].at[idxv.at[b, row]], bufs[b][i],
                                  gsem).wait()

    def fire_writes(b, g):
        tok = pl.multiple_of(g * CHUNK, CHUNK)
        for i, (_, col, w, _) in enumerate(FIELDS):
            pltpu.async_copy(
                bufs[b][i], out_hbm.at[pl.ds(tok, CHUNK), pl.ds(col, w)],
                wsems[b])

    def wait_writes(b):
        for i, (_, col, w, _) in enumerate(FIELDS):
            pltpu.make_async_copy(
                bufs[b][i], out_hbm.at[pl.ds(0, CHUNK), pl.ds(col, w)],
                wsems[b]).wait()

    pltpu.sync_copy(idx_hbm.at[chunk0], idxv.at[0])
    fire_gathers(0, chunk0)

    def pair(jj, carry):
        for b in (0, 1):
            j = jj * 2 + b
            g = chunk0 + j

            @pl.when(j + 1 < CPW)
            def _prefetch_idx():
                pltpu.async_copy(idx_hbm.at[g + 1], idxv.at[1 - b], isem)

            wait_gathers(b)
            fire_writes(b, g)

            @pl.when(j + 1 < CPW)
            def _next_gathers():
                pltpu.make_async_copy(idx_hbm.at[g + 1], idxv.at[1 - b],
                                      isem).wait()

                @pl.when(j >= 1)
                def _():
                    wait_writes(1 - b)

                fire_gathers(1 - b, g + 1)

        return carry

    lax.fori_loop(0, CPW // 2, pair, 0)
    wait_writes((CPW - 1) % 2)


@jax.jit
def _run(idx, sp, mv, ab, it, lm):
    def field_bufs():
        return tuple(pltpu.VMEM((CHUNK, w), jnp.float32) for _, _, w, _ in FIELDS)
    scratch = [
        pltpu.VMEM((2, 8, CHUNK), jnp.int32),
        field_bufs(),
        field_bufs(),
        pltpu.SemaphoreType.DMA,
        pltpu.SemaphoreType.DMA,
        pltpu.SemaphoreType.DMA,
        pltpu.SemaphoreType.DMA,
    ]
    kern = pl.kernel(
        _body,
        out_type=jax.ShapeDtypeStruct((TOKENS, D_OUT), jnp.float32),
        mesh=plsc.VectorSubcoreMesh(core_axis_name="c", subcore_axis_name="s"),
        scratch_types=scratch,
        compiler_params=pltpu.CompilerParams(use_tc_tiling_on_sc=False),
    )
    return kern(idx, sp, mv, ab, it, lm)


def kernel(int_ids, species_emb, move_emb, ability_emb, item_emb, last_move_emb):
    ids = int_ids.astype(jnp.int32)
    idx = ids.reshape(NCHUNKS, CHUNK, 8).transpose(0, 2, 1)
    out = _run(idx, species_emb, move_emb, ability_emb, item_emb, last_move_emb)
    return out.reshape(BATCH, N_TOKENS, D_OUT)


# speedup vs baseline: 9.9780x; 1.0123x over previous
import jax
import jax.numpy as jnp
from jax import lax
from jax.experimental import pallas as pl
from jax.experimental.pallas import tpu as pltpu
from jax.experimental.pallas import tpu_sc as plsc

BATCH = 4096
N_TOKENS = 50
TOKENS = BATCH * N_TOKENS
CHUNK = 128
NCHUNKS = TOKENS // CHUNK
NW = 32
CPW = NCHUNKS // NW
D_OUT = 256
N_ROWS = 100000

FIELDS = (
    (0, 0, 64, 0),
    (1, 64, 32, 1),
    (2, 96, 32, 1),
    (3, 128, 32, 1),
    (4, 160, 32, 1),
    (5, 192, 16, 2),
    (6, 208, 16, 3),
    (7, 224, 32, 4),
)
TABLE_WIDTHS = (64, 32, 16, 16, 32)


def _body(idx_hbm, sp_hbm, mv_hbm, ab_hbm, it_hbm, lm_hbm, out_hbm,
          idxv, bufs0, bufs1, gsem0, gsem1, wsem0, wsem1, isem):
    tables = (sp_hbm, mv_hbm, ab_hbm, it_hbm, lm_hbm)
    bufs = (bufs0, bufs1)
    gsems = (gsem0, gsem1)
    wsems = (wsem0, wsem1)
    wid = lax.axis_index("s") * 2 + lax.axis_index("c")
    chunk0 = wid * CPW

    def fire_gathers(b):
        for i, (row, _, _, targ) in enumerate(FIELDS):
            pltpu.async_copy(tables[targ].at[idxv.at[b, row]], bufs[b][i],
                             gsems[b])

    def wait_gathers(b):
        for i, (row, _, _, targ) in enumerate(FIELDS):
            pltpu.make_async_copy(tables[targ].at[idxv.at[b, row]], bufs[b][i],
                                  gsems[b]).wait()

    def fire_writes(b, g):
        tok = pl.multiple_of(g * CHUNK, CHUNK)
        for i, (_, col, w, _) in enumerate(FIELDS):
            pltpu.async_copy(
                bufs[b][i], out_hbm.at[pl.ds(tok, CHUNK), pl.ds(col, w)],
                wsems[b])

    def wait_writes(b):
        for i, (_, col, w, _) in enumerate(FIELDS):
            pltpu.make_async_copy(
                bufs[b][i], out_hbm.at[pl.ds(0, CHUNK), pl.ds(col, w)],
                wsems[b]).wait()

    pltpu.sync_copy(idx_hbm.at[chunk0], idxv.at[0])
    fire_gathers(0)

    def pair(jj, carry):
        for b in (0, 1):
            j = jj * 2 + b
            g = chunk0 + j

            @pl.when(j + 1 < CPW)
            def _prefetch_idx():
                pltpu.async_copy(idx_hbm.at[g + 1], idxv.at[1 - b], isem)

            @pl.when(j >= 1)
            def _():
                wait_writes(1 - b)

            @pl.when(j + 1 < CPW)
            def _next_gathers():
                pltpu.make_async_copy(idx_hbm.at[g + 1], idxv.at[1 - b],
                                      isem).wait()
                fire_gathers(1 - b)

            wait_gathers(b)
            fire_writes(b, g)

        return carry

    lax.fori_loop(0, CPW // 2, pair, 0)
    wait_writes((CPW - 1) % 2)


@jax.jit
def _run(idx, sp, mv, ab, it, lm):
    def field_bufs():
        return tuple(pltpu.VMEM((CHUNK, w), jnp.float32) for _, _, w, _ in FIELDS)
    scratch = [
        pltpu.VMEM((2, 8, CHUNK), jnp.int32),
        field_bufs(),
        field_bufs(),
        pltpu.SemaphoreType.DMA,
        pltpu.SemaphoreType.DMA,
        pltpu.SemaphoreType.DMA,
        pltpu.SemaphoreType.DMA,
        pltpu.SemaphoreType.DMA,
    ]
    kern = pl.kernel(
        _body,
        out_type=jax.ShapeDtypeStruct((TOKENS, D_OUT), jnp.float32),
        mesh=plsc.VectorSubcoreMesh(core_axis_name="c", subcore_axis_name="s"),
        scratch_types=scratch,
        compiler_params=pltpu.CompilerParams(use_tc_tiling_on_sc=False),
    )
    return kern(idx, sp, mv, ab, it, lm)


def kernel(int_ids, species_emb, move_emb, ability_emb, item_emb, last_move_emb):
    ids = int_ids.astype(jnp.int32)
    idx = ids.reshape(NCHUNKS, CHUNK, 8).transpose(0, 2, 1)
    flat = jnp.concatenate([
        species_emb.reshape(-1), move_emb.reshape(-1), ability_emb.reshape(-1),
        item_emb.reshape(-1), last_move_emb.reshape(-1)])
    views = []
    off = 0
    for w in TABLE_WIDTHS:
        views.append(lax.slice(flat, (off,), (off + N_ROWS * w,))
                     .reshape(N_ROWS, w))
        off += N_ROWS * w
    out = _run(idx, *views)
    return out.reshape(BATCH, N_TOKENS, D_OUT)
